# Initial kernel scaffold; baseline (speedup 1.0000x reference)
import jax
import jax.numpy as jnp
from jax.experimental import pallas as pl


def kernel(x, edge_attr, enc_W, enc_b, bond_W, bond_b, eps, mlp_W1, mlp_b1, mlp_g1, mlp_be1, mlp_W2, mlp_b2, bn_g, bn_b, vn_W1, vn_b1, vn_g1, vn_be1, vn_W2, vn_b2, vn_g2, vn_be2, head_W, head_b, edge_index, batch):
    raise NotImplementedError("write your pallas kernel here")


# speedup vs baseline: 2.5112x; 2.5112x over previous
import functools

import jax
import jax.numpy as jnp
from jax import lax
from jax.experimental import pallas as pl
from jax.experimental.pallas import tpu as pltpu
from jax.experimental.pallas import tpu_sc as plsc

F32 = jnp.float32
N_, E_, D_, DE_, L_, G_, T_ = 10000, 320000, 128, 16, 5, 64, 128
NB_, Nb_ = 10, 1000
EB_, Eb_ = 160, 2000
C_ = 128
NCHUNK_ = E_ // C_
RPT_ = 624
ZR_ = 208
REM_ = N_ - 16 * RPT_


def _dotbf(a, b):
    return jnp.dot(a.astype(jnp.bfloat16), b.astype(jnp.bfloat16),
                   preferred_element_type=F32)


def _encode(x, W, b):
    def body(x_ref, w_ref, b_ref, o_ref):
        o_ref[...] = _dotbf(x_ref[...], w_ref[...]) + b_ref[...]
    return pl.pallas_call(
        body,
        grid=(NB_,),
        in_specs=[pl.BlockSpec((Nb_, D_), lambda i: (i, 0)),
                  pl.BlockSpec((D_, D_), lambda i: (0, 0)),
                  pl.BlockSpec((1, D_), lambda i: (0, 0))],
        out_specs=pl.BlockSpec((Nb_, D_), lambda i: (i, 0)),
        out_shape=jax.ShapeDtypeStruct((N_, D_), F32),
    )(x, W, b)


def _edge_embed(edge_attr, bond_W, bond_b):
    def body(ea_ref, w_ref, b_ref, o_ref):
        o_ref[0] = _dotbf(ea_ref[...], w_ref[0]) + b_ref[0]
    return pl.pallas_call(
        body,
        grid=(L_, EB_),
        in_specs=[pl.BlockSpec((Eb_, DE_), lambda l, e: (e, 0)),
                  pl.BlockSpec((1, DE_, D_), lambda l, e: (l, 0, 0)),
                  pl.BlockSpec((1, 1, D_), lambda l, e: (l, 0, 0))],
        out_specs=pl.BlockSpec((1, Eb_, D_), lambda l, e: (l, e, 0)),
        out_shape=jax.ShapeDtypeStruct((L_, E_, D_), F32),
    )(edge_attr, bond_W, bond_b)


def _onehot(bt_ref):
    bt = bt_ref[0].reshape(Nb_, 1)
    return (bt == lax.broadcasted_iota(jnp.int32, (Nb_, G_), 1)).astype(F32)


def _colvar(y, st, W):
    def body(y_ref, st_ref, o_ref):
        i = pl.program_id(0)
        m = st_ref[0:1, :] * (1.0 / N_)
        d = y_ref[...] - m

        @pl.when(i == 0)
        def _():
            o_ref[...] = jnp.zeros_like(o_ref)
        o_ref[...] += jnp.sum(d * d, axis=0, keepdims=True)
    return pl.pallas_call(
        body,
        grid=(NB_,),
        in_specs=[pl.BlockSpec((Nb_, W), lambda i: (i, 0)),
                  pl.BlockSpec((2, W), lambda i: (0, 0))],
        out_specs=pl.BlockSpec((1, W), lambda i: (0, 0)),
        out_shape=jax.ShapeDtypeStruct((1, W), F32),
    )(y, st)


def _bn_from_stats(v_blk, st_ref, ssc_ref, g, b):
    m = st_ref[0:1, :] * (1.0 / N_)
    v = ssc_ref[0:1, :] * (1.0 / N_)
    return g * (v_blk - m) * lax.rsqrt(v + 1e-5) + b


def _hin(lprev, z2, st2, ssc2, bn_g, bn_b, vn, batch3):
    def body(z_ref, st_ref, ssc_ref, g_ref, b_ref, vn_ref, bt_ref, o_ref):
        h = jnp.maximum(_bn_from_stats(z_ref[...], st_ref, ssc_ref, g_ref[0],
                                       b_ref[0]), 0.0)
        oh = _onehot(bt_ref)
        o_ref[...] = h + jnp.dot(oh, vn_ref[...], preferred_element_type=F32, precision=lax.Precision.HIGHEST)
    return pl.pallas_call(
        body,
        grid=(NB_,),
        in_specs=[pl.BlockSpec((Nb_, D_), lambda i: (i, 0)),
                  pl.BlockSpec((2, D_), lambda i: (0, 0)),
                  pl.BlockSpec((1, D_), lambda i: (0, 0)),
                  pl.BlockSpec((1, 1, D_), lambda i: (lprev, 0, 0)),
                  pl.BlockSpec((1, 1, D_), lambda i: (lprev, 0, 0)),
                  pl.BlockSpec((G_, D_), lambda i: (0, 0)),
                  pl.BlockSpec((1, 1, Nb_), lambda i: (i, 0, 0))],
        out_specs=pl.BlockSpec((Nb_, D_), lambda i: (i, 0)),
        out_shape=jax.ShapeDtypeStruct((N_, D_), F32),
    )(z2, st2, ssc2, bn_g, bn_b, vn, batch3)


def _mlp1(l, h_in, agg, epsb, W1, b1):
    def body(h_ref, a_ref, e_ref, w_ref, b_ref, y_ref, st_ref):
        i = pl.program_id(0)
        z = e_ref[...] * h_ref[...] + a_ref[0] + a_ref[1]
        y = _dotbf(z, w_ref[0]) + b_ref[0]
        y_ref[...] = y

        @pl.when(i == 0)
        def _():
            st_ref[...] = jnp.zeros_like(st_ref)
        st_ref[0:1, :] += jnp.sum(y, axis=0, keepdims=True)
        st_ref[1:2, :] += jnp.sum(y * y, axis=0, keepdims=True)
    return pl.pallas_call(
        body,
        grid=(NB_,),
        in_specs=[pl.BlockSpec((Nb_, D_), lambda i: (i, 0)),
                  pl.BlockSpec((2, Nb_, D_), lambda i: (0, i, 0)),
                  pl.BlockSpec((1, D_), lambda i: (0, 0)),
                  pl.BlockSpec((1, D_, 2 * D_), lambda i: (l, 0, 0)),
                  pl.BlockSpec((1, 1, 2 * D_), lambda i: (l, 0, 0))],
        out_specs=[pl.BlockSpec((Nb_, 2 * D_), lambda i: (i, 0)),
                   pl.BlockSpec((2, 2 * D_), lambda i: (0, 0))],
        out_shape=[jax.ShapeDtypeStruct((N_, 2 * D_), F32),
                   jax.ShapeDtypeStruct((2, 2 * D_), F32)],
    )(h_in, agg, epsb, W1, b1)


def _mlp2(l, y1, st1, ssc1, g1, be1, W2, b2):
    def body(y_ref, st1_ref, ssc_ref, g_ref, be_ref, w_ref, b_ref, z_ref,
             st2_ref):
        i = pl.program_id(0)
        yn = jnp.maximum(_bn_from_stats(y_ref[...], st1_ref, ssc_ref,
                                        g_ref[0], be_ref[0]), 0.0)
        z = _dotbf(yn, w_ref[0]) + b_ref[0]
        z_ref[...] = z

        @pl.when(i == 0)
        def _():
            st2_ref[...] = jnp.zeros_like(st2_ref)
        st2_ref[0:1, :] += jnp.sum(z, axis=0, keepdims=True)
        st2_ref[1:2, :] += jnp.sum(z * z, axis=0, keepdims=True)
    return pl.pallas_call(
        body,
        grid=(NB_,),
        in_specs=[pl.BlockSpec((Nb_, 2 * D_), lambda i: (i, 0)),
                  pl.BlockSpec((2, 2 * D_), lambda i: (0, 0)),
                  pl.BlockSpec((1, 2 * D_), lambda i: (0, 0)),
                  pl.BlockSpec((1, 1, 2 * D_), lambda i: (l, 0, 0)),
                  pl.BlockSpec((1, 1, 2 * D_), lambda i: (l, 0, 0)),
                  pl.BlockSpec((1, 2 * D_, D_), lambda i: (l, 0, 0)),
                  pl.BlockSpec((1, 1, D_), lambda i: (l, 0, 0))],
        out_specs=[pl.BlockSpec((Nb_, D_), lambda i: (i, 0)),
                   pl.BlockSpec((2, D_), lambda i: (0, 0))],
        out_shape=[jax.ShapeDtypeStruct((N_, D_), F32),
                   jax.ShapeDtypeStruct((2, D_), F32)],
    )(y1, st1, ssc1, g1, be1, W2, b2)


def _bn_local(y, g, b):
    m = jnp.mean(y, axis=0, keepdims=True)
    d = y - m
    v = jnp.mean(d * d, axis=0, keepdims=True)
    return g * d * lax.rsqrt(v + 1e-5) + b


def _vn_update(l, h_in, batch3, vn, W1, b1, g1, be1, W2, b2, g2, be2):
    def body(h_ref, bt_ref, vn_ref, w1_ref, b1_ref, g1_ref, be1_ref,
             w2_ref, b2_ref, g2_ref, be2_ref, o_ref):
        i = pl.program_id(0)
        oh = _onehot(bt_ref)

        @pl.when(i == 0)
        def _():
            o_ref[...] = jnp.zeros_like(o_ref)
        o_ref[...] += jnp.dot(oh.T, h_ref[...], preferred_element_type=F32, precision=lax.Precision.HIGHEST)

        @pl.when(i == NB_ - 1)
        def _():
            vt = o_ref[...] + vn_ref[...]
            y = _dotbf(vt, w1_ref[0]) + b1_ref[0]
            y = jnp.maximum(_bn_local(y, g1_ref[0], be1_ref[0]), 0.0)
            y = _dotbf(y, w2_ref[0]) + b2_ref[0]
            y = jnp.maximum(_bn_local(y, g2_ref[0], be2_ref[0]), 0.0)
            o_ref[...] = y
    return pl.pallas_call(
        body,
        grid=(NB_,),
        in_specs=[pl.BlockSpec((Nb_, D_), lambda i: (i, 0)),
                  pl.BlockSpec((1, 1, Nb_), lambda i: (i, 0, 0)),
                  pl.BlockSpec((G_, D_), lambda i: (0, 0)),
                  pl.BlockSpec((1, D_, 2 * D_), lambda i: (l, 0, 0)),
                  pl.BlockSpec((1, 1, 2 * D_), lambda i: (l, 0, 0)),
                  pl.BlockSpec((1, 1, 2 * D_), lambda i: (l, 0, 0)),
                  pl.BlockSpec((1, 1, 2 * D_), lambda i: (l, 0, 0)),
                  pl.BlockSpec((1, 2 * D_, D_), lambda i: (l, 0, 0)),
                  pl.BlockSpec((1, 1, D_), lambda i: (l, 0, 0)),
                  pl.BlockSpec((1, 1, D_), lambda i: (l, 0, 0)),
                  pl.BlockSpec((1, 1, D_), lambda i: (l, 0, 0))],
        out_specs=pl.BlockSpec((G_, D_), lambda i: (0, 0)),
        out_shape=jax.ShapeDtypeStruct((G_, D_), F32),
    )(h_in, batch3, vn, W1, b1, g1, be1, W2, b2, g2, be2)


def _readout(z2, st2, ssc2, bn_g, bn_b, batch3, head_W, head_b):
    def body(z_ref, st_ref, ssc_ref, g_ref, b_ref, bt_ref, hw_ref, hb_ref,
             o_ref, acc, cnt):
        i = pl.program_id(0)
        h = _bn_from_stats(z_ref[...], st_ref, ssc_ref, g_ref[0], b_ref[0])
        oh = _onehot(bt_ref)

        @pl.when(i == 0)
        def _():
            acc[...] = jnp.zeros_like(acc)
            cnt[...] = jnp.zeros_like(cnt)
        acc[...] += jnp.dot(oh.T, h, preferred_element_type=F32, precision=lax.Precision.HIGHEST)
        cnt[...] += jnp.dot(oh.T, jnp.ones((Nb_, D_), F32),
                            preferred_element_type=F32, precision=lax.Precision.HIGHEST)

        @pl.when(i == NB_ - 1)
        def _():
            hg = acc[...] / jnp.maximum(cnt[...], 1.0)
            o_ref[...] = _dotbf(hg, hw_ref[...]) + hb_ref[...]
    return pl.pallas_call(
        body,
        grid=(NB_,),
        in_specs=[pl.BlockSpec((Nb_, D_), lambda i: (i, 0)),
                  pl.BlockSpec((2, D_), lambda i: (0, 0)),
                  pl.BlockSpec((1, D_), lambda i: (0, 0)),
                  pl.BlockSpec((1, 1, D_), lambda i: (L_ - 1, 0, 0)),
                  pl.BlockSpec((1, 1, D_), lambda i: (L_ - 1, 0, 0)),
                  pl.BlockSpec((1, 1, Nb_), lambda i: (i, 0, 0)),
                  pl.BlockSpec((D_, T_), lambda i: (0, 0)),
                  pl.BlockSpec((1, T_), lambda i: (0, 0))],
        out_specs=pl.BlockSpec((G_, T_), lambda i: (0, 0)),
        out_shape=jax.ShapeDtypeStruct((G_, T_), F32),
        scratch_shapes=[pltpu.VMEM((G_, D_), F32),
                        pltpu.VMEM((G_, D_), F32)],
    )(z2, st2, ssc2, bn_g, bn_b, batch3, head_W, head_b)


def _make_sc_msg(l):
    mesh = plsc.VectorSubcoreMesh(core_axis_name="c", subcore_axis_name="s")

    @functools.partial(
        pl.kernel,
        mesh=mesh,
        out_type=jax.ShapeDtypeStruct((2, N_, D_), F32),
        scratch_types=[
            pltpu.VMEM((C_,), jnp.int32),
            pltpu.VMEM((C_,), jnp.int32),
            pltpu.VMEM((C_, D_), F32),
            pltpu.VMEM((C_, D_), F32),
            pltpu.VMEM_SHARED((N_, D_), F32),
            pltpu.SemaphoreType.DMA,
        ],
    )
    def k(hin, ee, src, dst, out, src_v, dst_v, rows_v, ee_v, agg, sem):
        c = lax.axis_index("c")
        s = lax.axis_index("s")
        wid = c * 16 + s

        def zrow(i, carry):
            for j in range(8):
                rows_v[i, pl.ds(j * 16, 16)] = jnp.zeros((16,), F32)
            return carry
        lax.fori_loop(0, C_, zrow, 0)
        for kk in range(4):
            pltpu.sync_copy(rows_v, agg.at[pl.ds(s * RPT_ + kk * C_, C_), :])
        pltpu.sync_copy(rows_v.at[pl.ds(0, RPT_ - 4 * C_), :],
                        agg.at[pl.ds(s * RPT_ + 4 * C_, RPT_ - 4 * C_), :])

        @pl.when(s == 0)
        def _():
            pltpu.sync_copy(rows_v.at[pl.ds(0, REM_), :],
                            agg.at[pl.ds(16 * RPT_, REM_), :])
        plsc.subcore_barrier()

        nch = 78 + (wid < 4).astype(jnp.int32)

        def chunk(t, carry):
            base = (wid + 32 * t) * C_
            pltpu.sync_copy(src.at[pl.ds(base, C_)], src_v)
            pltpu.sync_copy(dst.at[pl.ds(base, C_)], dst_v)
            cp = pltpu.async_copy(hin.at[src_v], rows_v, sem)
            pltpu.sync_copy(ee.at[l, pl.ds(base, C_), :], ee_v)
            cp.wait()

            def erow(i, cc):
                for j in range(8):
                    sl = pl.ds(j * 16, 16)
                    rows_v[i, sl] = jnp.maximum(rows_v[i, sl] + ee_v[i, sl],
                                                0.0)
                return cc
            lax.fori_loop(0, C_, erow, 0)
            pltpu.sync_copy(rows_v, agg.at[dst_v], add=True)
            return carry
        lax.fori_loop(0, nch, chunk, 0)

        plsc.subcore_barrier()
        for kk in range(RPT_ // ZR_):
            r0 = s * RPT_ + kk * ZR_
            pltpu.sync_copy(agg.at[pl.ds(r0, ZR_), :],
                            out.at[c, pl.ds(r0, ZR_), :])

        @pl.when(s == 0)
        def _():
            pltpu.sync_copy(agg.at[pl.ds(16 * RPT_, REM_), :],
                            out.at[c, pl.ds(16 * RPT_, REM_), :])
    return k


def kernel(x, edge_attr, enc_W, enc_b, bond_W, bond_b, eps, mlp_W1, mlp_b1,
           mlp_g1, mlp_be1, mlp_W2, mlp_b2, bn_g, bn_b, vn_W1, vn_b1, vn_g1,
           vn_be1, vn_W2, vn_b2, vn_g2, vn_be2, head_W, head_b, edge_index,
           batch):
    src = edge_index[0].astype(jnp.int32)
    dst = edge_index[1].astype(jnp.int32)
    batch3 = batch.astype(jnp.int32).reshape(NB_, 1, Nb_)

    ee = _edge_embed(edge_attr, bond_W, bond_b.reshape(L_, 1, D_))
    bn_g3 = bn_g.reshape(L_, 1, D_)
    bn_b3 = bn_b.reshape(L_, 1, D_)
    mlp_b13 = mlp_b1.reshape(L_, 1, 2 * D_)
    mlp_g13 = mlp_g1.reshape(L_, 1, 2 * D_)
    mlp_be13 = mlp_be1.reshape(L_, 1, 2 * D_)
    mlp_b23 = mlp_b2.reshape(L_, 1, D_)
    vn_b13 = vn_b1.reshape(L_ - 1, 1, 2 * D_)
    vn_g13 = vn_g1.reshape(L_ - 1, 1, 2 * D_)
    vn_be13 = vn_be1.reshape(L_ - 1, 1, 2 * D_)
    vn_b23 = vn_b2.reshape(L_ - 1, 1, D_)
    vn_g23 = vn_g2.reshape(L_ - 1, 1, D_)
    vn_be23 = vn_be2.reshape(L_ - 1, 1, D_)
    h_in = _encode(x, enc_W, enc_b.reshape(1, D_))
    vn = jnp.zeros((G_, D_), F32)

    z2 = st2 = ssc2 = None
    for l in range(L_):
        if l > 0:
            h_in = _hin(l - 1, z2, st2, ssc2, bn_g3, bn_b3, vn, batch3)
        agg = _make_sc_msg(l)(h_in, ee, src, dst)
        if l < L_ - 1:
            vn = _vn_update(l, h_in, batch3, vn, vn_W1, vn_b13, vn_g13,
                            vn_be13, vn_W2, vn_b23, vn_g23, vn_be23)
        epsb = jnp.full((1, D_), 1.0, F32) + eps[l]
        y1, st1 = _mlp1(l, h_in, agg, epsb, mlp_W1, mlp_b13)
        ssc1 = _colvar(y1, st1, 2 * D_)
        z2, st2 = _mlp2(l, y1, st1, ssc1, mlp_g13, mlp_be13, mlp_W2, mlp_b23)
        ssc2 = _colvar(z2, st2, D_)

    return _readout(z2, st2, ssc2, bn_g3, bn_b3, batch3, head_W,
                    head_b.reshape(1, T_))
